# Initial kernel scaffold; baseline (speedup 1.0000x reference)
import jax
import jax.numpy as jnp
from jax.experimental import pallas as pl


def kernel(workclass, education, marital_status, occupation, relationship, race, sex, native_country, mean_tables, std_tables, fc_weights, action_emb, rand_array):
    raise NotImplementedError("write your pallas kernel here")


# speedup vs baseline: 19.7640x; 19.7640x over previous
import functools

import jax
import jax.numpy as jnp
from jax import lax
from jax.experimental import pallas as pl
from jax.experimental.pallas import tpu as pltpu
from jax.experimental.pallas import tpu_sc as plsc

_COLS = 8
_EMB = 12
_D = 16
_B = 16384
_NPAIR = _COLS * (_COLS - 1) // 2


def _softplus_prep(sd_ref, out_ref):
    out_ref[...] = 0.01 * jnp.log(1.0 + jnp.exp(sd_ref[...]))


def kernel(workclass, education, marital_status, occupation, relationship,
           race, sex, native_country, mean_tables, std_tables, fc_weights,
           action_emb, rand_array):
    cols = [workclass, education, marital_status, occupation, relationship,
            race, sex, native_country]

    mu96 = mean_tables.reshape(_COLS * _EMB * _D)
    s96 = pl.pallas_call(
        _softplus_prep,
        out_shape=jax.ShapeDtypeStruct((_COLS * _EMB, _D), jnp.float32),
    )(std_tables.reshape(_COLS * _EMB, _D)).reshape(_COLS * _EMB * _D)

    info = plsc.get_sparse_core_info()
    nc = info.num_cores
    nw = info.num_cores * info.num_subcores
    rpw = _B // nw
    ngrp = rpw // _D

    mesh = plsc.VectorSubcoreMesh(core_axis_name="c", subcore_axis_name="s")

    @functools.partial(
        pl.kernel,
        out_type=jax.ShapeDtypeStruct((_B, 2), jnp.float32),
        mesh=mesh,
        compiler_params=pltpu.CompilerParams(needs_layout_passes=False),
        scratch_types=[
            pltpu.VMEM((_COLS, rpw), jnp.int32),
            pltpu.VMEM((rpw * _D,), jnp.float32),
            pltpu.VMEM((_COLS * _EMB * _D,), jnp.float32),
            pltpu.VMEM((_COLS * _EMB * _D,), jnp.float32),
            pltpu.VMEM((_NPAIR, _D), jnp.float32),
            pltpu.VMEM((2, _D), jnp.float32),
            pltpu.VMEM((rpw, 2), jnp.float32),
            pltpu.SemaphoreType.DMA,
        ],
    )
    def _fm_sc(c0, c1, c2, c3, c4, c5, c6, c7, mu_ref, s_ref, w_ref, a_ref,
               rnd_ref, out_ref, idx_b, v_b, mu_v, s_v, w_b, a_b, o_b, sem):
        crefs = [c0, c1, c2, c3, c4, c5, c6, c7]
        wid = lax.axis_index("s") * nc + lax.axis_index("c")
        base = wid * rpw

        descs = [
            pltpu.async_copy(mu_ref, mu_v, sem),
            pltpu.async_copy(s_ref, s_v, sem),
            pltpu.async_copy(w_ref, w_b, sem),
            pltpu.async_copy(a_ref, a_b, sem),
            pltpu.async_copy(rnd_ref.at[pl.ds(base * _D, rpw * _D)], v_b, sem),
        ]
        descs += [
            pltpu.async_copy(crefs[i].at[pl.ds(base, rpw)], idx_b.at[i], sem)
            for i in range(_COLS)
        ]
        for d in descs:
            d.wait()

        for i in range(_COLS):
            off = jnp.full((_D,), _EMB * _D * i, jnp.int32)
            for t in range(rpw // _D):
                sl = pl.ds(t * _D, _D)
                idx_b[i, sl] = (idx_b[i, sl] * _D) + off

        w = [w_b[k] for k in range(_NPAIR)]
        a0 = a_b[0]
        a1 = a_b[1]
        lanes = jnp.arange(_D, dtype=jnp.int32)
        m15 = lanes == (_D - 1)
        col0 = jnp.zeros((_D,), jnp.int32)
        col1 = jnp.full((_D,), 1, jnp.int32)

        @plsc.parallel_loop(0, ngrp)
        def _grp(g):
            g16 = g * _D
            ivecs = [idx_b[i, pl.ds(g16, _D)] for i in range(_COLS)]
            for rl in range(_D):
                rlvec = jnp.full((_D,), rl, jnp.int32)
                e = []
                for i in range(_COLS):
                    addr = jnp.take_along_axis(ivecs[i], rlvec, axis=0) + lanes
                    mu_i = plsc.load_gather(mu_v, [addr])
                    s_i = plsc.load_gather(s_v, [addr])
                    e.append((mu_i, s_i))
                v = v_b[pl.ds((g16 + rl) * _D, _D)]
                e = [mu_i + s_i * v for (mu_i, s_i) in e]
                k = 0
                acc = None
                for i in range(_COLS - 1):
                    gg = w[k] * e[i + 1]
                    k += 1
                    for j in range(i + 2, _COLS):
                        gg = gg + w[k] * e[j]
                        k += 1
                    t = e[i] * gg
                    acc = t if acc is None else acc + t
                se = e[0]
                for i in range(1, _COLS):
                    se = se + e[i]
                z0 = acc + se * a0
                z1 = acc + se * a1
                cz0 = plsc.cumsum(z0)
                cz1 = plsc.cumsum(z1)
                ridx = jnp.full((_D,), g16 + rl, jnp.int32)
                plsc.store_scatter(o_b, [ridx, col0], cz0, mask=m15)
                plsc.store_scatter(o_b, [ridx, col1], cz1, mask=m15)

        pltpu.sync_copy(o_b, out_ref.at[pl.ds(base, rpw)])

    return _fm_sc(*cols, mu96, s96, fc_weights, action_emb, rand_array)
